# baseline (speedup 1.0000x reference)
import jax
import jax.numpy as jnp
from jax import lax
from jax.experimental import pallas as pl
from jax.experimental.pallas import tpu as pltpu

T, NS, NW, NQ, D = 32, 320, 16, 240, 512
TB = 2


def _body(lab_ref, sup_ref, qry_ref, out_ref):
    for b in range(TB):
        lab = lab_ref[b]
        sup = sup_ref[b]
        q = qry_ref[b]

        wids = lax.broadcasted_iota(jnp.int32, (NW, NS), 0)
        oh = jnp.where(wids == lab, 1.0, 0.0).astype(jnp.float32)
        cnt = jnp.sum(oh, axis=1, keepdims=True)
        psum = lax.dot_general(
            oh, sup, (((1,), (0,)), ((), ())),
            preferred_element_type=jnp.float32,
            precision=lax.Precision.HIGHEST)
        protos = psum / cnt

        wn = jnp.sum(protos * protos, axis=1, keepdims=True)
        scores = lax.dot_general(
            protos, q, (((1,), (1,)), ((), ())),
            preferred_element_type=jnp.float32,
            precision=lax.Precision.HIGHEST)
        d2 = wn - 2.0 * scores

        idx = lax.broadcasted_iota(jnp.int32, (NW, NQ), 0)
        m = jnp.min(d2, axis=0, keepdims=True)
        pred = jnp.min(jnp.where(d2 == m, idx, NW), axis=0, keepdims=True)
        out_ref[b] = pred.astype(jnp.int32)


@jax.jit
def kernel(support_features, support_labels, query_features):
    labels3 = support_labels.reshape(T, 1, NS)
    out = pl.pallas_call(
        _body,
        grid=(T // TB,),
        in_specs=[
            pl.BlockSpec((TB, 1, NS), lambda t: (t, 0, 0)),
            pl.BlockSpec((TB, NS, D), lambda t: (t, 0, 0)),
            pl.BlockSpec((TB, NQ, D), lambda t: (t, 0, 0)),
        ],
        out_specs=pl.BlockSpec((TB, 1, NQ), lambda t: (t, 0, 0)),
        out_shape=jax.ShapeDtypeStruct((T, 1, NQ), jnp.int32),
        compiler_params=pltpu.CompilerParams(
            dimension_semantics=("parallel",)),
    )(labels3, support_features, query_features)
    return out.reshape(T, NQ)


# speedup vs baseline: 1.0773x; 1.0773x over previous
import jax
import jax.numpy as jnp
from jax import lax
from jax.experimental import pallas as pl
from jax.experimental.pallas import tpu as pltpu

T, NS, NW, NQ, D = 32, 320, 16, 240, 512
TB = 4


def _body(lab_ref, sup_ref, qry_ref, out_ref):
    for b in range(TB):
        lab = lab_ref[b]
        sup = sup_ref[b]
        q = qry_ref[b]

        wids = lax.broadcasted_iota(jnp.int32, (NW, NS), 0)
        oh = jnp.where(wids == lab, 1.0, 0.0).astype(jnp.float32)
        cnt = jnp.sum(oh, axis=1, keepdims=True)
        psum = lax.dot_general(
            oh, sup, (((1,), (0,)), ((), ())),
            preferred_element_type=jnp.float32,
            precision=lax.Precision.HIGHEST)
        protos = psum / cnt

        wn = jnp.sum(protos * protos, axis=1, keepdims=True)
        scores = lax.dot_general(
            protos, q, (((1,), (1,)), ((), ())),
            preferred_element_type=jnp.float32,
            precision=lax.Precision.HIGHEST)
        d2 = wn - 2.0 * scores

        idx = lax.broadcasted_iota(jnp.int32, (NW, NQ), 0)
        m = jnp.min(d2, axis=0, keepdims=True)
        pred = jnp.min(jnp.where(d2 == m, idx, NW), axis=0, keepdims=True)
        out_ref[b] = pred.astype(jnp.int32)


@jax.jit
def kernel(support_features, support_labels, query_features):
    labels3 = support_labels.reshape(T, 1, NS)
    out = pl.pallas_call(
        _body,
        grid=(T // TB,),
        in_specs=[
            pl.BlockSpec((TB, 1, NS), lambda t: (t, 0, 0)),
            pl.BlockSpec((TB, NS, D), lambda t: (t, 0, 0)),
            pl.BlockSpec((TB, NQ, D), lambda t: (t, 0, 0)),
        ],
        out_specs=pl.BlockSpec((TB, 1, NQ), lambda t: (t, 0, 0)),
        out_shape=jax.ShapeDtypeStruct((T, 1, NQ), jnp.int32),
        compiler_params=pltpu.CompilerParams(
            dimension_semantics=("parallel",)),
    )(labels3, support_features, query_features)
    return out.reshape(T, NQ)
